# baseline (speedup 1.0000x reference)
import functools

import jax
import jax.numpy as jnp
from jax import lax
from jax.experimental import pallas as pl
from jax.experimental.pallas import tpu as pltpu
from jax.experimental.pallas import tpu_sc as plsc

BATCH = 16384
EMB_D = 32

NUM_CORES = 2
NUM_SUBCORES = 16
NW = NUM_CORES * NUM_SUBCORES
B_PER_W = BATCH // NW
CHUNK = 128
NCHUNK = B_PER_W // CHUNK


def _sc_gather_body(users_hbm, items_hbm, gu_t, gi_t, mu_t, mi_t,
                    gu_o, gi_o, mu_o, mi_o,
                    idx_u, idx_i, ru, ri, rmu, rmi, sem):
    wid = lax.axis_index("s") * NUM_CORES + lax.axis_index("c")
    base = wid * B_PER_W
    row0 = wid * NCHUNK
    pltpu.sync_copy(users_hbm.at[pl.ds(row0, NCHUNK)], idx_u)
    pltpu.sync_copy(items_hbm.at[pl.ds(row0, NCHUNK)], idx_i)
    copies = []
    for c in range(NCHUNK):
        dst = pl.ds(c * CHUNK, CHUNK)
        copies.append(pltpu.async_copy(gu_t.at[idx_u.at[c]], ru.at[dst], sem))
        copies.append(pltpu.async_copy(gi_t.at[idx_i.at[c]], ri.at[dst], sem))
        copies.append(pltpu.async_copy(mu_t.at[idx_u.at[c]], rmu.at[dst], sem))
        copies.append(pltpu.async_copy(mi_t.at[idx_i.at[c]], rmi.at[dst], sem))
    for cp in copies:
        cp.wait()
    out_slice = pl.ds(base, B_PER_W)
    pltpu.sync_copy(ru, gu_o.at[out_slice])
    pltpu.sync_copy(ri, gi_o.at[out_slice])
    pltpu.sync_copy(rmu, mu_o.at[out_slice])
    pltpu.sync_copy(rmi, mi_o.at[out_slice])


@jax.jit
def _sc_gather(users, items, gu_t, gi_t, mu_t, mi_t):
    emb = jax.ShapeDtypeStruct((BATCH, EMB_D), jnp.float32)
    mesh = plsc.VectorSubcoreMesh(core_axis_name="c", subcore_axis_name="s")
    f = pl.kernel(
        _sc_gather_body,
        out_type=(emb, emb, emb, emb),
        mesh=mesh,
        compiler_params=pltpu.CompilerParams(use_tc_tiling_on_sc=False),
        scratch_types=[
            pltpu.VMEM((NCHUNK, CHUNK), jnp.int32),
            pltpu.VMEM((NCHUNK, CHUNK), jnp.int32),
            pltpu.VMEM((B_PER_W, EMB_D), jnp.float32),
            pltpu.VMEM((B_PER_W, EMB_D), jnp.float32),
            pltpu.VMEM((B_PER_W, EMB_D), jnp.float32),
            pltpu.VMEM((B_PER_W, EMB_D), jnp.float32),
            pltpu.SemaphoreType.DMA,
        ],
    )
    return f(users, items, gu_t, gi_t, mu_t, mi_t)


def _tc_dense_body(gu, gi, mu, mi, W1, b1, W2, b2, Wf, bf, out):
    prod = gu[...] * gi[...]
    x1 = (jnp.dot(mu[...], W1[0:32, :], preferred_element_type=jnp.float32)
          + jnp.dot(mi[...], W1[32:64, :], preferred_element_type=jnp.float32)
          + b1[...])
    h = jnp.maximum(x1, 0.0)
    m2 = jnp.dot(h, W2[...], preferred_element_type=jnp.float32) + b2[...]
    logit = (jnp.dot(prod, Wf[0:32, :], preferred_element_type=jnp.float32)
             + jnp.dot(m2, Wf[32:96, :], preferred_element_type=jnp.float32)
             + bf[...])
    out[...] = jax.nn.sigmoid(logit)


@jax.jit
def _tc_dense(gu, gi, mu, mi, W1, b1, W2, b2, Wf, bf):
    BLK = 4096
    grid = (BATCH // BLK,)
    emb_spec = pl.BlockSpec((BLK, EMB_D), lambda i: (i, 0))
    full = lambda shape: pl.BlockSpec(shape, lambda i: tuple(0 for _ in shape))
    return pl.pallas_call(
        _tc_dense_body,
        grid=grid,
        in_specs=[
            emb_spec, emb_spec, emb_spec, emb_spec,
            full((64, 128)), full((128,)), full((128, 64)), full((64,)),
            full((96, 1)), full((1,)),
        ],
        out_specs=pl.BlockSpec((BLK, 1), lambda i: (i, 0)),
        out_shape=jax.ShapeDtypeStruct((BATCH, 1), jnp.float32),
    )(gu, gi, mu, mi, W1, b1, W2, b2, Wf, bf)


def kernel(X, gmf_user_emb, gmf_item_emb, mlp_user_emb, mlp_item_emb,
           W1, b1, W2, b2, Wf, bf):
    users = X[:, 0].astype(jnp.int32).reshape(NW * NCHUNK, CHUNK)
    items = X[:, 1].astype(jnp.int32).reshape(NW * NCHUNK, CHUNK)
    gu, gi, mu, mi = _sc_gather(users, items, gmf_user_emb, gmf_item_emb,
                                mlp_user_emb, mlp_item_emb)
    return _tc_dense(gu, gi, mu, mi, W1, b1, W2, b2, Wf, bf)
